# baseline (speedup 1.0000x reference)
import functools

import jax
import jax.numpy as jnp
from jax import lax
from jax.experimental import pallas as pl
from jax.experimental.pallas import tpu as pltpu
from jax.experimental.pallas import tpu_sc as plsc

_D = 1024
_B = 4
_S = 2048
_P = 20
_R = _S + _P
_NDT = _D // 128

_NC = 2
_NS = 16
_NW = _NC * _NS
_SEQ_W = _S // _NW
_CSEQ = 4
_NCHUNK = _SEQ_W // _CSEQ
_NBUF = 4
_CPIECE = _CSEQ * _B * _NDT
_SPROWS = _B * _P * _NDT
_OUTROWS = _R * _NDT * _B
_WROWS = (100000 // 8) * 8 * _NDT


def _embed_concat(tokens_flat, sp_pieces, w_pieces):
    mesh = plsc.VectorSubcoreMesh(core_axis_name="c", subcore_axis_name="s")

    @functools.partial(
        pl.kernel,
        mesh=mesh,
        out_type=jax.ShapeDtypeStruct((_OUTROWS, 128), jnp.float32),
        scratch_types=[
            pltpu.VMEM((8, 128), jnp.int32),
            pltpu.VMEM((_NBUF, _CPIECE, 128), jnp.float32),
            pltpu.VMEM((_SPROWS // 8, 128), jnp.float32),
            pltpu.SemaphoreType.DMA,
            pltpu.SemaphoreType.DMA,
            pltpu.SemaphoreType.DMA,
            pltpu.SemaphoreType.DMA,
            pltpu.SemaphoreType.DMA,
            pltpu.SemaphoreType.DMA,
            pltpu.SemaphoreType.DMA,
            pltpu.SemaphoreType.DMA,
            pltpu.SemaphoreType.DMA,
        ],
        compiler_params=pltpu.CompilerParams(needs_layout_passes=False),
    )
    def k(tok_hbm, sp_hbm, w_hbm, out_hbm, idx_v, rows_v, sp_v,
          gsem0, gsem1, gsem2, gsem3,
          osem0, osem1, osem2, osem3, psem):
        wid = lax.axis_index("s") * _NC + lax.axis_index("c")
        seq0 = pl.multiple_of(wid * _SEQ_W, _SEQ_W)
        iota16 = lax.iota(jnp.int32, 16)

        pltpu.sync_copy(
            tok_hbm.at[pl.ds(pl.multiple_of((wid // 4) * 8, 8), 8)],
            idx_v)
        row0 = (wid // 2) % 2 * 4
        col0 = wid % 2 * 64

        sp_off = pl.multiple_of(wid % 8 * (_SPROWS // 8), 8)

        @pl.when(wid < 8)
        def _():
            pltpu.async_copy(sp_hbm.at[pl.ds(sp_off, _SPROWS // 8)],
                             sp_v, psem)

        gsem = (gsem0, gsem1, gsem2, gsem3)
        osem = (osem0, osem1, osem2, osem3)

        def issue_gathers(c, p):
            for v in range(_CPIECE // 16):
                pp = 16 * v + iota16
                sl = lax.shift_right_logical(pp, 5)
                dt = lax.bitwise_and(lax.shift_right_logical(pp, 2), 7)
                bb = lax.bitwise_and(pp, 3)
                t = plsc.load_gather(
                    idx_v, [row0 + bb, col0 + c * _CSEQ + sl])
                gidx = (lax.shift_right_logical(t, 3) * (8 * _NDT)
                        + dt * 8 + lax.bitwise_and(t, 7))
                pltpu.async_copy(
                    w_hbm.at[gidx], rows_v.at[p, pl.ds(16 * v, 16)],
                    gsem[p])

        def drain_gathers(p):
            pltpu.make_async_copy(
                w_hbm.at[pl.ds(0, _CPIECE)], rows_v.at[p], gsem[p]).wait()

        def issue_write(c, p):
            row = pl.multiple_of(
                (_P + seq0 + c * _CSEQ) * (_NDT * _B), _CPIECE)
            pltpu.async_copy(
                rows_v.at[p], out_hbm.at[pl.ds(row, _CPIECE)], osem[p])

        def drain_write(p):
            pltpu.make_async_copy(
                rows_v.at[p], out_hbm.at[pl.ds(0, _CPIECE)], osem[p]).wait()

        def body(i2, carry):
            for h in range(_NBUF):
                g = _NBUF * i2 + h
                q = (h - 1) % _NBUF

                @pl.when(jnp.logical_and(g >= _NBUF,
                                         g < _NCHUNK + _NBUF))
                def _():
                    drain_write(h)

                @pl.when(g < _NCHUNK)
                def _():
                    issue_gathers(g, h)

                @pl.when(jnp.logical_and(g >= 1, g < _NCHUNK + 1))
                def _():
                    drain_gathers(q)
                    issue_write(g - 1, q)
            return carry

        nsteps = _NCHUNK + _NBUF
        lax.fori_loop(0, (nsteps + _NBUF - 1) // _NBUF, body, 0)

        @pl.when(wid < 8)
        def _():
            pltpu.make_async_copy(
                sp_hbm.at[pl.ds(sp_off, _SPROWS // 8)], sp_v, psem).wait()
            pltpu.async_copy(
                sp_v, out_hbm.at[pl.ds(sp_off, _SPROWS // 8)], psem).wait()

    return k(tokens_flat, sp_pieces, w_pieces)


def kernel(tokens, soft_prompts, W):
    tokens_flat = (tokens.astype(jnp.int32).reshape(_B, _S // 128, 128)
                   .transpose(1, 0, 2).reshape(_B * _S // 128, 128))
    sp_pieces = (soft_prompts.reshape(_B, _P, _NDT, 128)
                 .transpose(1, 2, 0, 3).reshape(_SPROWS, 128))
    w_pieces = (W.reshape(_WROWS // 64, 8, _NDT, 128)
                .transpose(0, 2, 1, 3).reshape(_WROWS, 128))
    out = _embed_concat(tokens_flat, sp_pieces, w_pieces)
    return (out.reshape(_R, _NDT, _B, 128)
            .transpose(2, 0, 1, 3).reshape(_B, _R, _D))


# speedup vs baseline: 1.0156x; 1.0156x over previous
import functools

import jax
import jax.numpy as jnp
from jax import lax
from jax.experimental import pallas as pl
from jax.experimental.pallas import tpu as pltpu
from jax.experimental.pallas import tpu_sc as plsc

_D = 1024
_B = 4
_S = 2048
_P = 20
_R = _S + _P
_NDT = _D // 128

_NC = 2
_NS = 16
_NW = _NC * _NS
_SEQ_W = _S // _NW
_CSEQ = 4
_NCHUNK = _SEQ_W // _CSEQ
_NBUF = 4
_CPIECE = _CSEQ * _B * _NDT
_SPROWS = _B * _P * _NDT
_OUTROWS = _R * _NDT * _B
_WROWS = (100000 // 8) * 8 * _NDT


def _embed_concat(tokens_flat, sp_pieces, w_pieces):
    mesh = plsc.VectorSubcoreMesh(core_axis_name="c", subcore_axis_name="s")

    @functools.partial(
        pl.kernel,
        mesh=mesh,
        out_type=jax.ShapeDtypeStruct((_OUTROWS, 128), jnp.float32),
        scratch_types=[
            pltpu.VMEM((8, 128), jnp.int32),
            pltpu.VMEM((_NBUF, _CPIECE, 128), jnp.float32),
            pltpu.VMEM((_SPROWS // 8, 128), jnp.float32),
            pltpu.SemaphoreType.DMA,
            pltpu.SemaphoreType.DMA,
            pltpu.SemaphoreType.DMA,
            pltpu.SemaphoreType.DMA,
            pltpu.SemaphoreType.DMA,
            pltpu.SemaphoreType.DMA,
            pltpu.SemaphoreType.DMA,
            pltpu.SemaphoreType.DMA,
            pltpu.SemaphoreType.DMA,
        ],
        compiler_params=pltpu.CompilerParams(needs_layout_passes=False),
    )
    def k(tok_hbm, sp_hbm, w_hbm, out_hbm, idx_v, rows_v, sp_v,
          gsem0, gsem1, gsem2, gsem3,
          osem0, osem1, osem2, osem3, psem):
        wid = lax.axis_index("s") * _NC + lax.axis_index("c")
        seq0 = pl.multiple_of(wid * _SEQ_W, _SEQ_W)
        iota16 = lax.iota(jnp.int32, 16)

        pltpu.sync_copy(
            tok_hbm.at[pl.ds(pl.multiple_of((wid // 4) * 8, 8), 8)],
            idx_v)
        row0 = (wid // 2) % 2 * 4
        col0 = wid % 2 * 64

        sp_off = pl.multiple_of(wid % 8 * (_SPROWS // 8), 8)

        @pl.when(wid < 8)
        def _():
            pltpu.async_copy(sp_hbm.at[pl.ds(sp_off, _SPROWS // 8)],
                             sp_v, psem)

        gsem = (gsem0, gsem1, gsem2, gsem3)
        osem = (osem0, osem1, osem2, osem3)

        def issue_gathers(c, p):
            for v in range(_CPIECE // 16):
                pp = 16 * v + iota16
                sl = lax.shift_right_logical(pp, 5)
                dt = lax.bitwise_and(lax.shift_right_logical(pp, 2), 7)
                bb = lax.bitwise_and(pp, 3)
                t = plsc.load_gather(
                    idx_v, [row0 + bb, col0 + c * _CSEQ + sl])
                gidx = (lax.shift_right_logical(t, 3) * (8 * _NDT)
                        + dt * 8 + lax.bitwise_and(t, 7))
                pltpu.async_copy(
                    w_hbm.at[gidx], rows_v.at[p, pl.ds(16 * v, 16)],
                    gsem[p])

        def drain_gathers(p):
            pltpu.make_async_copy(
                w_hbm.at[pl.ds(0, _CPIECE)], rows_v.at[p], gsem[p]).wait()

        def issue_write(c, p):
            row = pl.multiple_of(
                (_P + seq0 + c * _CSEQ) * (_NDT * _B), _CPIECE)
            pltpu.async_copy(
                rows_v.at[p], out_hbm.at[pl.ds(row, _CPIECE)], osem[p])

        def drain_write(p):
            pltpu.make_async_copy(
                rows_v.at[p], out_hbm.at[pl.ds(0, _CPIECE)], osem[p]).wait()

        def body(i2, carry):
            for h in range(_NBUF):
                g = _NBUF * i2 + h
                q = (h - 2) % _NBUF

                @pl.when(jnp.logical_and(g >= _NBUF,
                                         g < _NCHUNK + _NBUF))
                def _():
                    drain_write(h)

                @pl.when(g < _NCHUNK)
                def _():
                    issue_gathers(g, h)

                @pl.when(jnp.logical_and(g >= 2, g < _NCHUNK + 2))
                def _():
                    drain_gathers(q)
                    issue_write(g - 2, q)
            return carry

        nsteps = _NCHUNK + _NBUF
        lax.fori_loop(0, (nsteps + _NBUF - 1) // _NBUF, body, 0)

        @pl.when(wid < 8)
        def _():
            pltpu.make_async_copy(
                sp_hbm.at[pl.ds(sp_off, _SPROWS // 8)], sp_v, psem).wait()
            pltpu.async_copy(
                sp_v, out_hbm.at[pl.ds(sp_off, _SPROWS // 8)], psem).wait()

    return k(tokens_flat, sp_pieces, w_pieces)


def kernel(tokens, soft_prompts, W):
    tokens_flat = (tokens.astype(jnp.int32).reshape(_B, _S // 128, 128)
                   .transpose(1, 0, 2).reshape(_B * _S // 128, 128))
    sp_pieces = (soft_prompts.reshape(_B, _P, _NDT, 128)
                 .transpose(1, 2, 0, 3).reshape(_SPROWS, 128))
    w_pieces = (W.reshape(_WROWS // 64, 8, _NDT, 128)
                .transpose(0, 2, 1, 3).reshape(_WROWS, 128))
    out = _embed_concat(tokens_flat, sp_pieces, w_pieces)
    return (out.reshape(_R, _NDT, _B, 128)
            .transpose(2, 0, 1, 3).reshape(_B, _R, _D))
